# baseline (speedup 1.0000x reference)
import jax
import jax.numpy as jnp
from jax.experimental import pallas as pl
from jax.experimental.pallas import tpu as pltpu

E, N, D, DFF = 8, 2048, 768, 3072
BT = 512


def _mlp_kernel(x_ref, w1_ref, b1_ref, w2_ref, b2_ref, o_ref):
    x = x_ref[0]
    h = jnp.dot(x, w1_ref[0], preferred_element_type=jnp.float32)
    h = jax.nn.gelu(h + b1_ref[0]).astype(jnp.bfloat16)
    o = jnp.dot(h, w2_ref[0], preferred_element_type=jnp.float32)
    o_ref[0] = o + b2_ref[0]


def kernel(x, W1, b1, W2, b2):
    B = x.shape[0]
    xe = x.reshape(E, N, D).astype(jnp.bfloat16)
    W1 = W1.astype(jnp.bfloat16)
    W2 = W2.astype(jnp.bfloat16)
    b1r = b1.reshape(E, 1, DFF)
    b2r = b2.reshape(E, 1, D)

    out = pl.pallas_call(
        _mlp_kernel,
        grid=(E, N // BT),
        in_specs=[
            pl.BlockSpec((1, BT, D), lambda e, t: (e, t, 0)),
            pl.BlockSpec((1, D, DFF), lambda e, t: (e, 0, 0)),
            pl.BlockSpec((1, 1, DFF), lambda e, t: (e, 0, 0)),
            pl.BlockSpec((1, DFF, D), lambda e, t: (e, 0, 0)),
            pl.BlockSpec((1, 1, D), lambda e, t: (e, 0, 0)),
        ],
        out_specs=pl.BlockSpec((1, BT, D), lambda e, t: (e, t, 0)),
        out_shape=jax.ShapeDtypeStruct((E, N, D), jnp.float32),
        compiler_params=pltpu.CompilerParams(
            dimension_semantics=("arbitrary", "arbitrary"),
        ),
    )(xe, W1, b1r, W2, b2r)

    return out.reshape(B, E, N, D)


# speedup vs baseline: 1.3963x; 1.3963x over previous
import jax
import jax.numpy as jnp
from jax.experimental import pallas as pl
from jax.experimental.pallas import tpu as pltpu

E, N, D, DFF = 8, 2048, 768, 3072
BT = 512


def _mlp_kernel(x_ref, w1_ref, b1_ref, w2_ref, b2_ref, o_ref):
    x = x_ref[0]
    h = jnp.dot(x, w1_ref[0], preferred_element_type=jnp.float32)
    h = jax.nn.gelu(h + b1_ref[0])
    o = jnp.dot(h, w2_ref[0], preferred_element_type=jnp.float32)
    o_ref[0] = o + b2_ref[0]


def kernel(x, W1, b1, W2, b2):
    B = x.shape[0]
    xe = x.reshape(E, N, D)
    b1r = b1.reshape(E, 1, DFF)
    b2r = b2.reshape(E, 1, D)

    out = pl.pallas_call(
        _mlp_kernel,
        grid=(E, N // BT),
        in_specs=[
            pl.BlockSpec((1, BT, D), lambda e, t: (e, t, 0)),
            pl.BlockSpec((1, D, DFF), lambda e, t: (e, 0, 0)),
            pl.BlockSpec((1, 1, DFF), lambda e, t: (e, 0, 0)),
            pl.BlockSpec((1, DFF, D), lambda e, t: (e, 0, 0)),
            pl.BlockSpec((1, 1, D), lambda e, t: (e, 0, 0)),
        ],
        out_specs=pl.BlockSpec((1, BT, D), lambda e, t: (e, t, 0)),
        out_shape=jax.ShapeDtypeStruct((E, N, D), jnp.float32),
        compiler_params=pltpu.CompilerParams(
            dimension_semantics=("parallel", "parallel"),
        ),
    )(xe, W1, b1r, W2, b2r)

    return out.reshape(B, E, N, D)


# speedup vs baseline: 1.5248x; 1.0920x over previous
import jax
import jax.numpy as jnp
from jax.experimental import pallas as pl
from jax.experimental.pallas import tpu as pltpu

E, N, D, DFF = 8, 2048, 768, 3072
BT = 1024
FC = 768


def _mlp_kernel(x_ref, w1_ref, b1_ref, w2_ref, b2_ref, o_ref):
    x = x_ref[0]
    acc = jnp.broadcast_to(b2_ref[0], (BT, D))
    for f in range(DFF // FC):
        lo, hi = f * FC, (f + 1) * FC
        h = jnp.dot(x, w1_ref[0, :, lo:hi], preferred_element_type=jnp.float32)
        h = jax.nn.gelu(h + b1_ref[0, :, lo:hi])
        acc = acc + jnp.dot(h, w2_ref[0, lo:hi, :],
                            preferred_element_type=jnp.float32)
    o_ref[0] = acc


def kernel(x, W1, b1, W2, b2):
    B = x.shape[0]
    xe = x.reshape(E, N, D)
    b1r = b1.reshape(E, 1, DFF)
    b2r = b2.reshape(E, 1, D)

    out = pl.pallas_call(
        _mlp_kernel,
        grid=(E, N // BT),
        in_specs=[
            pl.BlockSpec((1, BT, D), lambda e, t: (e, t, 0)),
            pl.BlockSpec((1, D, DFF), lambda e, t: (e, 0, 0)),
            pl.BlockSpec((1, 1, DFF), lambda e, t: (e, 0, 0)),
            pl.BlockSpec((1, DFF, D), lambda e, t: (e, 0, 0)),
            pl.BlockSpec((1, 1, D), lambda e, t: (e, 0, 0)),
        ],
        out_specs=pl.BlockSpec((1, BT, D), lambda e, t: (e, t, 0)),
        out_shape=jax.ShapeDtypeStruct((E, N, D), jnp.float32),
        compiler_params=pltpu.CompilerParams(
            dimension_semantics=("parallel", "parallel"),
        ),
    )(xe, W1, b1r, W2, b2r)

    return out.reshape(B, E, N, D)
